# baseline (speedup 1.0000x reference)
import functools

import jax
import jax.numpy as jnp
from jax import lax
from jax.experimental import pallas as pl
from jax.experimental.pallas import tpu as pltpu
from jax.experimental.pallas import tpu_sc as plsc

N = 10000
D = 128
DW = D // 2
E = 320000
L = 16
NC, NS = 2, 16
NW = NC * NS
EW = E // NW
CB = 16
NCH = EW // CB
NBUF = 5

_GDN = lax.GatherDimensionNumbers(
    offset_dims=(), collapsed_slice_dims=(0,), start_index_map=(0,)
)


def _lane_shuffle(v, perm):
    return lax.gather(
        v, perm.reshape(L, 1), _GDN, (1,),
        mode=lax.GatherScatterMode.PROMISE_IN_BOUNDS,
    )


def _pack_pair(acc_e, acc_o):
    lo = lax.bitcast_convert_type(acc_e.astype(jnp.bfloat16), jnp.int16)
    hi = lax.bitcast_convert_type(acc_o.astype(jnp.bfloat16), jnp.int16)
    lo32 = jnp.bitwise_and(lo.astype(jnp.int32), 0xFFFF)
    hi32 = lax.shift_left(hi.astype(jnp.int32), 16)
    return jnp.bitwise_or(lo32, hi32)


def _table_body(x_ref, wae_ref, wao_ref, wbe_ref, wbo_ref, be_ref, bo_ref, o_ref):
    x = x_ref[...]
    dn = (((1,), (1,)), ((), ()))
    acc_ae = lax.dot_general(x, wae_ref[...], dn, preferred_element_type=jnp.float32)
    acc_ao = lax.dot_general(x, wao_ref[...], dn, preferred_element_type=jnp.float32)
    acc_be = lax.dot_general(x, wbe_ref[...], dn, preferred_element_type=jnp.float32)
    acc_bo = lax.dot_general(x, wbo_ref[...], dn, preferred_element_type=jnp.float32)
    wa = _pack_pair(acc_ae + be_ref[...], acc_ao + bo_ref[...])
    wb = _pack_pair(acc_be, acc_bo)
    o_ref[...] = jnp.concatenate([wa, wb], axis=1)


def _build_tables(x, W1, b1):
    BN = 2000
    NB = N // BN
    return pl.pallas_call(
        _table_body,
        grid=(NB,),
        in_specs=[
            pl.BlockSpec((BN, D), lambda i: (i, 0)),
            pl.BlockSpec((DW, D), lambda i: (0, 0)),
            pl.BlockSpec((DW, D), lambda i: (0, 0)),
            pl.BlockSpec((DW, D), lambda i: (0, 0)),
            pl.BlockSpec((DW, D), lambda i: (0, 0)),
            pl.BlockSpec((1, DW), lambda i: (0, 0)),
            pl.BlockSpec((1, DW), lambda i: (0, 0)),
        ],
        out_specs=pl.BlockSpec((BN, D), lambda i: (i, 0)),
        out_shape=jax.ShapeDtypeStruct((N, D), jnp.int32),
    )(x, W1[0::2, :D], W1[1::2, :D], W1[0::2, D:], W1[1::2, D:],
      b1[0::2].reshape(1, DW), b1[1::2].reshape(1, DW))


def _sc_body(t_hbm, idx_hbm, b2_hbm, w2bf_hbm, out_hbm,
             idx_v, rows, out_v, b2_v, w2bf_v, ts, sems):
    cid = lax.axis_index("c")
    sid = lax.axis_index("s")
    wid = sid * NC + cid
    base = wid * EW

    rows_per = N // NS
    pltpu.sync_copy(
        t_hbm.at[pl.ds(sid * rows_per, rows_per), pl.ds(0, DW)],
        ts.at[pl.ds(sid * rows_per, rows_per)],
    )
    pltpu.sync_copy(
        t_hbm.at[pl.ds(sid * rows_per, rows_per), pl.ds(DW, DW)],
        ts.at[pl.ds(N + sid * rows_per, rows_per)],
    )

    pltpu.sync_copy(idx_hbm.at[pl.ds(base * 2, 2 * EW)], idx_v)
    pltpu.sync_copy(b2_hbm, b2_v)
    pltpu.sync_copy(w2bf_hbm, w2bf_v)
    w2v = [w2bf_v[pl.ds(k * 2 * L, 2 * L)] for k in range(D // (2 * L))]
    b2v = b2_v[...]
    lanes = lax.iota(jnp.int32, L)
    zero_bf = jnp.zeros((2 * L,), jnp.bfloat16)
    plsc.subcore_barrier()

    def start(c, b):
        pltpu.async_copy(ts.at[idx_v.at[pl.ds(c * 2 * CB, 2 * CB)]], rows[b], sems[b])

    def wait(c, b):
        pltpu.make_async_copy(
            ts.at[idx_v.at[pl.ds(c * 2 * CB, 2 * CB)]], rows[b], sems[b]
        ).wait()

    def _split(p):
        v = plsc.bitcast(p, jnp.int32)
        lo = plsc.bitcast(lax.shift_left(v, jnp.int32(16)), jnp.float32)
        hi = plsc.bitcast(lax.bitwise_and(v, jnp.int32(-65536)), jnp.float32)
        return lo, hi

    def compute(c, b):
        def edge(e, res):
            ps = []
            for k in range(D // (2 * L)):
                wa = rows[b][e, pl.ds(k * L, L)]
                wb = rows[b][CB + e, pl.ds(k * L, L)]
                s = plsc.bitcast(wa, jnp.bfloat16) + plsc.bitcast(wb, jnp.bfloat16)
                ps.append(jnp.maximum(s, zero_bf) * w2v[k])
            l0, h0 = _split(ps[0] + ps[1])
            l1, h1 = _split(ps[2] + ps[3])
            acc = (l0 + h0) + (l1 + h1)
            for sh in (8, 4, 2, 1):
                acc = acc + _lane_shuffle(acc, lanes ^ sh)
            return jnp.where(lanes == e, acc, res)

        res = lax.fori_loop(0, CB, edge, jnp.zeros((L,), jnp.float32), unroll=4)
        out_v[pl.ds(c * CB, CB)] = res + b2v

    for b in range(NBUF):
        start(b, b)

    def ring(p, _):
        for b in range(NBUF):
            c = p * NBUF + b
            wait(c, b)
            compute(c, b)

            @pl.when(c + NBUF < NCH)
            def _():
                start(c + NBUF, b)

        return 0

    lax.fori_loop(0, NCH // NBUF, ring, 0)
    pltpu.sync_copy(out_v, out_hbm.at[pl.ds(base, EW)])


@functools.partial(
    pl.kernel,
    out_type=jax.ShapeDtypeStruct((E,), jnp.float32),
    mesh=plsc.VectorSubcoreMesh(
        core_axis_name="c", subcore_axis_name="s", num_cores=NC, num_subcores=NS
    ),
    compiler_params=pltpu.CompilerParams(
        needs_layout_passes=False, use_tc_tiling_on_sc=False
    ),
    scratch_types=[
        pltpu.VMEM((2 * EW,), jnp.int32),
    ] + [pltpu.VMEM((2 * CB, DW), jnp.int32) for _ in range(NBUF)] + [
        pltpu.VMEM((EW,), jnp.float32),
        pltpu.VMEM((L,), jnp.float32),
        pltpu.VMEM((D,), jnp.bfloat16),
        pltpu.VMEM_SHARED((2 * N, DW), jnp.int32),
    ] + [pltpu.SemaphoreType.DMA for _ in range(NBUF)],
)
def _sc_edge_head(t_hbm, idx_hbm, b2_hbm, w2bf_hbm, out_hbm, idx_v,
                  r0, r1, r2, r3, r4,
                  out_v, b2_v, w2bf_v, ts, s0, s1, s2, s3, s4):
    _sc_body(t_hbm, idx_hbm, b2_hbm, w2bf_hbm, out_hbm, idx_v,
             [r0, r1, r2, r3, r4],
             out_v, b2_v, w2bf_v, ts, [s0, s1, s2, s3, s4])


def kernel(x, edge_index, edge_attr, edge_label_index, W1, b1, W2, b2):
    tables = _build_tables(x, W1, b1)
    src3 = edge_label_index[0].reshape(NW, NCH, L)
    dst3 = edge_label_index[1].reshape(NW, NCH, L) + N
    idx2 = jnp.stack([src3, dst3], axis=2).reshape(-1)
    b2l = jnp.broadcast_to(b2, (L,))
    w2bf = W2.reshape(-1).astype(jnp.bfloat16)
    pred = _sc_edge_head(tables, idx2, b2l, w2bf)
    return (pred, x)


# speedup vs baseline: 1.8065x; 1.8065x over previous
import functools

import jax
import jax.numpy as jnp
from jax import lax
from jax.experimental import pallas as pl
from jax.experimental.pallas import tpu as pltpu
from jax.experimental.pallas import tpu_sc as plsc

N = 10000
D = 128
DW = D // 2
E = 320000
L = 16
NC, NS = 2, 16
NW = NC * NS
EW = E // NW
CB = 16
NCH = EW // CB
NBUF = 5

_GDN = lax.GatherDimensionNumbers(
    offset_dims=(), collapsed_slice_dims=(0,), start_index_map=(0,)
)


def _lane_shuffle(v, perm):
    return lax.gather(
        v, perm.reshape(L, 1), _GDN, (1,),
        mode=lax.GatherScatterMode.PROMISE_IN_BOUNDS,
    )


def _pack_pair(acc_e, acc_o):
    lo = lax.bitcast_convert_type(acc_e.astype(jnp.bfloat16), jnp.int16)
    hi = lax.bitcast_convert_type(acc_o.astype(jnp.bfloat16), jnp.int16)
    lo32 = jnp.bitwise_and(lo.astype(jnp.int32), 0xFFFF)
    hi32 = lax.shift_left(hi.astype(jnp.int32), 16)
    return jnp.bitwise_or(lo32, hi32)


def _table_body(x_ref, wae_ref, wao_ref, wbe_ref, wbo_ref, be_ref, bo_ref, o_ref):
    x = x_ref[...]
    dn = (((1,), (1,)), ((), ()))
    acc_ae = lax.dot_general(x, wae_ref[...], dn, preferred_element_type=jnp.float32)
    acc_ao = lax.dot_general(x, wao_ref[...], dn, preferred_element_type=jnp.float32)
    acc_be = lax.dot_general(x, wbe_ref[...], dn, preferred_element_type=jnp.float32)
    acc_bo = lax.dot_general(x, wbo_ref[...], dn, preferred_element_type=jnp.float32)
    wa = _pack_pair(acc_ae + be_ref[...], acc_ao + bo_ref[...])
    wb = _pack_pair(acc_be, acc_bo)
    o_ref[...] = jnp.concatenate([wa, wb], axis=1)


def _build_tables(x, W1, b1):
    BN = 2000
    NB = N // BN
    return pl.pallas_call(
        _table_body,
        grid=(NB,),
        in_specs=[
            pl.BlockSpec((BN, D), lambda i: (i, 0)),
            pl.BlockSpec((DW, D), lambda i: (0, 0)),
            pl.BlockSpec((DW, D), lambda i: (0, 0)),
            pl.BlockSpec((DW, D), lambda i: (0, 0)),
            pl.BlockSpec((DW, D), lambda i: (0, 0)),
            pl.BlockSpec((1, DW), lambda i: (0, 0)),
            pl.BlockSpec((1, DW), lambda i: (0, 0)),
        ],
        out_specs=pl.BlockSpec((BN, D), lambda i: (i, 0)),
        out_shape=jax.ShapeDtypeStruct((N, D), jnp.int32),
    )(x, W1[0::2, :D], W1[1::2, :D], W1[0::2, D:], W1[1::2, D:],
      b1[0::2].reshape(1, DW), b1[1::2].reshape(1, DW))


def _sc_body(t_hbm, src_hbm, dst_hbm, b2_hbm, w2bf_hbm, out_hbm,
             idx_s, idx_d, rows_a, rows_b, out_v, b2_v, w2bf_v, ts, sems):
    cid = lax.axis_index("c")
    sid = lax.axis_index("s")
    wid = sid * NC + cid
    base = wid * EW

    rows_per = N // NS
    pltpu.sync_copy(
        t_hbm.at[pl.ds(sid * rows_per, rows_per), pl.ds(0, DW)],
        ts.at[pl.ds(sid * rows_per, rows_per)],
    )
    pltpu.sync_copy(
        t_hbm.at[pl.ds(sid * rows_per, rows_per), pl.ds(DW, DW)],
        ts.at[pl.ds(N + sid * rows_per, rows_per)],
    )

    pltpu.sync_copy(src_hbm.at[pl.ds(base, EW)], idx_s)
    pltpu.sync_copy(dst_hbm.at[pl.ds(base, EW)], idx_d)
    pltpu.sync_copy(b2_hbm, b2_v)
    pltpu.sync_copy(w2bf_hbm, w2bf_v)
    w2v = [w2bf_v[pl.ds(k * 2 * L, 2 * L)] for k in range(D // (2 * L))]
    b2v = b2_v[...]
    lanes = lax.iota(jnp.int32, L)
    zero_bf = jnp.zeros((2 * L,), jnp.bfloat16)
    plsc.subcore_barrier()

    def start(c, b):
        pltpu.async_copy(ts.at[idx_s.at[pl.ds(c * CB, CB)]], rows_a[b], sems[b])
        pltpu.async_copy(ts.at[idx_d.at[pl.ds(c * CB, CB)]], rows_b[b], sems[b])

    def wait(c, b):
        pltpu.make_async_copy(
            ts.at[idx_s.at[pl.ds(c * CB, CB)]], rows_a[b], sems[b]
        ).wait()
        pltpu.make_async_copy(
            ts.at[idx_d.at[pl.ds(c * CB, CB)]], rows_b[b], sems[b]
        ).wait()

    def _split(p):
        v = plsc.bitcast(p, jnp.int32)
        lo = plsc.bitcast(lax.shift_left(v, jnp.int32(16)), jnp.float32)
        hi = plsc.bitcast(lax.bitwise_and(v, jnp.int32(-65536)), jnp.float32)
        return lo, hi

    def compute(c, b):
        res = b2v
        for e in range(CB):
            ps = []
            for k in range(D // (2 * L)):
                wa = rows_a[b][e, pl.ds(k * L, L)]
                wb = rows_b[b][e, pl.ds(k * L, L)]
                s = plsc.bitcast(wa, jnp.bfloat16) + plsc.bitcast(wb, jnp.bfloat16)
                ps.append(jnp.maximum(s, zero_bf) * w2v[k])
            l0, h0 = _split(ps[0] + ps[1])
            l1, h1 = _split(ps[2] + ps[3])
            acc = (l0 + h0) + (l1 + h1)
            for sh in (8, 4, 2, 1):
                acc = acc + _lane_shuffle(acc, lanes ^ sh)
            res = jnp.where(lanes == e, acc, res)
        out_v[pl.ds(c * CB, CB)] = res

    for b in range(NBUF):
        start(b, b)

    def ring(p, _):
        for b in range(NBUF):
            c = p * NBUF + b
            wait(c, b)
            compute(c, b)

            @pl.when(c + NBUF < NCH)
            def _():
                start(c + NBUF, b)

        return 0

    lax.fori_loop(0, NCH // NBUF, ring, 0)
    pltpu.sync_copy(out_v, out_hbm.at[pl.ds(base, EW)])


@functools.partial(
    pl.kernel,
    out_type=jax.ShapeDtypeStruct((E,), jnp.float32),
    mesh=plsc.VectorSubcoreMesh(
        core_axis_name="c", subcore_axis_name="s", num_cores=NC, num_subcores=NS
    ),
    compiler_params=pltpu.CompilerParams(
        needs_layout_passes=False, use_tc_tiling_on_sc=False
    ),
    scratch_types=[
        pltpu.VMEM((EW,), jnp.int32),
        pltpu.VMEM((EW,), jnp.int32),
    ] + [pltpu.VMEM((CB, DW), jnp.int32) for _ in range(2 * NBUF)] + [
        pltpu.VMEM((EW,), jnp.float32),
        pltpu.VMEM((L,), jnp.float32),
        pltpu.VMEM((D,), jnp.bfloat16),
        pltpu.VMEM_SHARED((2 * N, DW), jnp.int32),
    ] + [pltpu.SemaphoreType.DMA for _ in range(NBUF)],
)
def _sc_edge_head(t_hbm, src_hbm, dst_hbm, b2_hbm, w2bf_hbm, out_hbm, idx_s, idx_d,
                  a0, a1, a2, a3, a4, b0, b1_, b2_, b3, b4,
                  out_v, b2_v, w2bf_v, ts, s0, s1, s2, s3, s4):
    _sc_body(t_hbm, src_hbm, dst_hbm, b2_hbm, w2bf_hbm, out_hbm, idx_s, idx_d,
             [a0, a1, a2, a3, a4], [b0, b1_, b2_, b3, b4],
             out_v, b2_v, w2bf_v, ts, [s0, s1, s2, s3, s4])


def kernel(x, edge_index, edge_attr, edge_label_index, W1, b1, W2, b2):
    tables = _build_tables(x, W1, b1)
    src = edge_label_index[0]
    dstN = edge_label_index[1] + N
    b2l = jnp.broadcast_to(b2, (L,))
    w2bf = W2.reshape(-1).astype(jnp.bfloat16)
    pred = _sc_edge_head(tables, src, dstN, b2l, w2bf)
    return (pred, x)


# speedup vs baseline: 1.8483x; 1.0231x over previous
import functools

import jax
import jax.numpy as jnp
from jax import lax
from jax.experimental import pallas as pl
from jax.experimental.pallas import tpu as pltpu
from jax.experimental.pallas import tpu_sc as plsc

N = 10000
D = 128
DW = D // 2
E = 320000
L = 16
NC, NS = 2, 16
NW = NC * NS
EW = E // NW
CB = 16
NCH = EW // CB
NBUF = 5

_GDN = lax.GatherDimensionNumbers(
    offset_dims=(), collapsed_slice_dims=(0,), start_index_map=(0,)
)


def _lane_shuffle(v, perm):
    return lax.gather(
        v, perm.reshape(L, 1), _GDN, (1,),
        mode=lax.GatherScatterMode.PROMISE_IN_BOUNDS,
    )


def _pack_pair(acc_e, acc_o):
    lo = lax.bitcast_convert_type(acc_e.astype(jnp.bfloat16), jnp.int16)
    hi = lax.bitcast_convert_type(acc_o.astype(jnp.bfloat16), jnp.int16)
    lo32 = jnp.bitwise_and(lo.astype(jnp.int32), 0xFFFF)
    hi32 = lax.shift_left(hi.astype(jnp.int32), 16)
    return jnp.bitwise_or(lo32, hi32)


def _table_body(x_ref, wae_ref, wao_ref, wbe_ref, wbo_ref, be_ref, bo_ref, o_ref):
    x = x_ref[...]
    dn = (((1,), (1,)), ((), ()))
    acc_ae = lax.dot_general(x, wae_ref[...], dn, preferred_element_type=jnp.float32)
    acc_ao = lax.dot_general(x, wao_ref[...], dn, preferred_element_type=jnp.float32)
    acc_be = lax.dot_general(x, wbe_ref[...], dn, preferred_element_type=jnp.float32)
    acc_bo = lax.dot_general(x, wbo_ref[...], dn, preferred_element_type=jnp.float32)
    wa = _pack_pair(acc_ae + be_ref[...], acc_ao + bo_ref[...])
    wb = _pack_pair(acc_be, acc_bo)
    o_ref[...] = jnp.concatenate([wa, wb], axis=1)


def _build_tables(x, W1, b1):
    BN = 2000
    NB = N // BN
    return pl.pallas_call(
        _table_body,
        grid=(NB,),
        in_specs=[
            pl.BlockSpec((BN, D), lambda i: (i, 0)),
            pl.BlockSpec((DW, D), lambda i: (0, 0)),
            pl.BlockSpec((DW, D), lambda i: (0, 0)),
            pl.BlockSpec((DW, D), lambda i: (0, 0)),
            pl.BlockSpec((DW, D), lambda i: (0, 0)),
            pl.BlockSpec((1, DW), lambda i: (0, 0)),
            pl.BlockSpec((1, DW), lambda i: (0, 0)),
        ],
        out_specs=pl.BlockSpec((BN, D), lambda i: (i, 0)),
        out_shape=jax.ShapeDtypeStruct((N, D), jnp.int32),
    )(x, W1[0::2, :D], W1[1::2, :D], W1[0::2, D:], W1[1::2, D:],
      b1[0::2].reshape(1, DW), b1[1::2].reshape(1, DW))


def _sc_body(t_hbm, src_hbm, dst_hbm, b2_hbm, w2bf_hbm, out_hbm,
             idx_s, idx_d, rows_a, rows_b, out_v, b2_v, w2bf_v, ts, sems):
    cid = lax.axis_index("c")
    sid = lax.axis_index("s")
    wid = sid * NC + cid
    base = wid * EW

    rows_per = N // NS
    pltpu.sync_copy(
        t_hbm.at[pl.ds(sid * rows_per, rows_per), pl.ds(0, DW)],
        ts.at[pl.ds(sid * rows_per, rows_per)],
    )
    pltpu.sync_copy(
        t_hbm.at[pl.ds(sid * rows_per, rows_per), pl.ds(DW, DW)],
        ts.at[pl.ds(N + sid * rows_per, rows_per)],
    )

    pltpu.sync_copy(src_hbm.at[pl.ds(base, EW)], idx_s)
    pltpu.sync_copy(dst_hbm.at[pl.ds(base, EW)], idx_d)
    pltpu.sync_copy(b2_hbm, b2_v)
    pltpu.sync_copy(w2bf_hbm, w2bf_v)
    w2v = [w2bf_v[pl.ds(k * 2 * L, 2 * L)] for k in range(D // (2 * L))]
    b2v = b2_v[...]
    lanes = lax.iota(jnp.int32, L)
    zero_bf = jnp.zeros((2 * L,), jnp.bfloat16)
    plsc.subcore_barrier()

    def start(c, b):
        pltpu.async_copy(ts.at[idx_s.at[pl.ds(c * CB, CB)]], rows_a[b], sems[b])
        pltpu.async_copy(ts.at[idx_d.at[pl.ds(c * CB, CB)]], rows_b[b], sems[b])

    def wait(c, b):
        pltpu.make_async_copy(
            ts.at[idx_s.at[pl.ds(c * CB, CB)]], rows_a[b], sems[b]
        ).wait()
        pltpu.make_async_copy(
            ts.at[idx_d.at[pl.ds(c * CB, CB)]], rows_b[b], sems[b]
        ).wait()

    def _split(p):
        v = plsc.bitcast(p, jnp.int32)
        lo = plsc.bitcast(lax.shift_left(v, jnp.int32(16)), jnp.float32)
        hi = plsc.bitcast(lax.bitwise_and(v, jnp.int32(-65536)), jnp.float32)
        return lo, hi

    def compute(c, b):
        accs = []
        for e in range(CB):
            ps = []
            for k in range(D // (2 * L)):
                wa = rows_a[b][e, pl.ds(k * L, L)]
                wb = rows_b[b][e, pl.ds(k * L, L)]
                s = plsc.bitcast(wa, jnp.bfloat16) + plsc.bitcast(wb, jnp.bfloat16)
                ps.append(jnp.maximum(s, zero_bf) * w2v[k])
            l0, h0 = _split(ps[0] + ps[1])
            l1, h1 = _split(ps[2] + ps[3])
            accs.append((l0 + h0) + (l1 + h1))
        vs = accs
        d = 1
        while len(vs) > 1:
            nxt = []
            mlo = (lanes & d) == 0
            for i in range(0, len(vs), 2):
                x, y = vs[i], vs[i + 1]
                sx = _lane_shuffle(x, lanes ^ d)
                sy = _lane_shuffle(y, lanes ^ d)
                nxt.append(jnp.where(mlo, x, sy) + jnp.where(mlo, sx, y))
            vs = nxt
            d *= 2
        out_v[pl.ds(c * CB, CB)] = vs[0] + b2v

    for b in range(NBUF):
        start(b, b)

    def ring(p, _):
        for b in range(NBUF):
            c = p * NBUF + b
            wait(c, b)
            compute(c, b)

            @pl.when(c + NBUF < NCH)
            def _():
                start(c + NBUF, b)

        return 0

    lax.fori_loop(0, NCH // NBUF, ring, 0)
    pltpu.sync_copy(out_v, out_hbm.at[pl.ds(base, EW)])


@functools.partial(
    pl.kernel,
    out_type=jax.ShapeDtypeStruct((E,), jnp.float32),
    mesh=plsc.VectorSubcoreMesh(
        core_axis_name="c", subcore_axis_name="s", num_cores=NC, num_subcores=NS
    ),
    compiler_params=pltpu.CompilerParams(
        needs_layout_passes=False, use_tc_tiling_on_sc=False
    ),
    scratch_types=[
        pltpu.VMEM((EW,), jnp.int32),
        pltpu.VMEM((EW,), jnp.int32),
    ] + [pltpu.VMEM((CB, DW), jnp.int32) for _ in range(2 * NBUF)] + [
        pltpu.VMEM((EW,), jnp.float32),
        pltpu.VMEM((L,), jnp.float32),
        pltpu.VMEM((D,), jnp.bfloat16),
        pltpu.VMEM_SHARED((2 * N, DW), jnp.int32),
    ] + [pltpu.SemaphoreType.DMA for _ in range(NBUF)],
)
def _sc_edge_head(t_hbm, src_hbm, dst_hbm, b2_hbm, w2bf_hbm, out_hbm, idx_s, idx_d,
                  a0, a1, a2, a3, a4, b0, b1_, b2_, b3, b4,
                  out_v, b2_v, w2bf_v, ts, s0, s1, s2, s3, s4):
    _sc_body(t_hbm, src_hbm, dst_hbm, b2_hbm, w2bf_hbm, out_hbm, idx_s, idx_d,
             [a0, a1, a2, a3, a4], [b0, b1_, b2_, b3, b4],
             out_v, b2_v, w2bf_v, ts, [s0, s1, s2, s3, s4])


def kernel(x, edge_index, edge_attr, edge_label_index, W1, b1, W2, b2):
    tables = _build_tables(x, W1, b1)
    src = edge_label_index[0]
    dstN = edge_label_index[1] + N
    b2l = jnp.broadcast_to(b2, (L,))
    w2bf = W2.reshape(-1).astype(jnp.bfloat16)
    pred = _sc_edge_head(tables, src, dstN, b2l, w2bf)
    return (pred, x)
